# Initial kernel scaffold; baseline (speedup 1.0000x reference)
import jax
import jax.numpy as jnp
from jax.experimental import pallas as pl


def kernel(x, edge_index, batch, W0, b0, W1, b1):
    raise NotImplementedError("write your pallas kernel here")


# speedup vs baseline: 15.5386x; 15.5386x over previous
import functools

import jax
import jax.numpy as jnp
from jax import lax
from jax.experimental import pallas as pl
from jax.experimental.pallas import tpu as pltpu
from jax.experimental.pallas import tpu_sc as plsc

NC = 2
NS = 16
CH = 128


def _make_deg(n_pad, cpt):
    rows_per_tile = n_pad // NS
    mesh = plsc.VectorSubcoreMesh(core_axis_name="c", subcore_axis_name="s")

    @functools.partial(
        pl.kernel,
        out_type=jax.ShapeDtypeStruct((NC, n_pad, 1), jnp.float32),
        mesh=mesh,
        scratch_types=[
            pltpu.VMEM((cpt, CH), jnp.int32),
            pltpu.VMEM((CH, 1), jnp.float32),
            pltpu.VMEM_SHARED((n_pad, 1), jnp.float32),
        ],
        compiler_params=pltpu.CompilerParams(use_tc_tiling_on_sc=False),
    )
    def deg_kernel(dst_hbm, ones_hbm, zeros_hbm, out_hbm, dst_v, ones_v, acc):
        c = lax.axis_index("c")
        s = lax.axis_index("s")
        wid = s * NC + c
        r0 = s * rows_per_tile
        pltpu.sync_copy(zeros_hbm.at[pl.ds(r0, rows_per_tile)],
                        acc.at[pl.ds(r0, rows_per_tile)])
        pltpu.sync_copy(dst_hbm.at[pl.ds(wid * cpt, cpt)], dst_v)
        pltpu.sync_copy(ones_hbm, ones_v)
        plsc.subcore_barrier()

        def body(j, carry):
            pltpu.sync_copy(ones_v, acc.at[dst_v.at[j]], add=True)
            return carry

        lax.fori_loop(0, cpt, body, 0)
        plsc.subcore_barrier()
        pltpu.sync_copy(acc.at[pl.ds(r0, rows_per_tile)],
                        out_hbm.at[c, pl.ds(r0, rows_per_tile)])

    return deg_kernel


def _make_agg(n_pad, cpt, width):
    rows_per_tile = n_pad // NS
    mesh = plsc.VectorSubcoreMesh(core_axis_name="c", subcore_axis_name="s")

    @functools.partial(
        pl.kernel,
        out_type=jax.ShapeDtypeStruct((NC, n_pad, width), jnp.float32),
        mesh=mesh,
        scratch_types=[
            pltpu.VMEM((cpt, CH), jnp.int32),
            pltpu.VMEM((cpt, CH), jnp.int32),
            pltpu.VMEM((CH, width), jnp.float32),
            pltpu.VMEM_SHARED((n_pad, width), jnp.float32),
        ],
        compiler_params=pltpu.CompilerParams(use_tc_tiling_on_sc=False),
    )
    def agg_kernel(table_hbm, src_hbm, dst_hbm, zeros_hbm, out_hbm,
                   src_v, dst_v, rows_v, acc):
        c = lax.axis_index("c")
        s = lax.axis_index("s")
        wid = s * NC + c
        r0 = s * rows_per_tile
        pltpu.sync_copy(zeros_hbm.at[pl.ds(r0, rows_per_tile)],
                        acc.at[pl.ds(r0, rows_per_tile)])
        pltpu.sync_copy(src_hbm.at[pl.ds(wid * cpt, cpt)], src_v)
        pltpu.sync_copy(dst_hbm.at[pl.ds(wid * cpt, cpt)], dst_v)
        plsc.subcore_barrier()

        def body(j, carry):
            pltpu.sync_copy(table_hbm.at[src_v.at[j]], rows_v)
            pltpu.sync_copy(rows_v, acc.at[dst_v.at[j]], add=True)
            return carry

        lax.fori_loop(0, cpt, body, 0)
        plsc.subcore_barrier()
        pltpu.sync_copy(acc.at[pl.ds(r0, rows_per_tile)],
                        out_hbm.at[c, pl.ds(r0, rows_per_tile)])

    return agg_kernel


def _mm0_body(x_ref, w_ref, p0_ref, p1_ref, y_ref, dinv_ref):
    deg = p0_ref[:] + p1_ref[:] + 1.0
    dinv = lax.rsqrt(deg)
    y = jnp.dot(x_ref[:], w_ref[:], preferred_element_type=jnp.float32)
    y_ref[:] = y * dinv
    dinv_ref[:] = dinv


def _mid_body(p0_ref, p1_ref, y0_ref, dinv_ref, b_ref, w_ref, out_ref):
    dinv = dinv_ref[:]
    pre = (p0_ref[:] + p1_ref[:] + y0_ref[:]) * dinv + b_ref[:]
    h = jnp.where(pre >= 0, pre, 0.01 * pre)
    out_ref[:] = jnp.dot(h, w_ref[:], preferred_element_type=jnp.float32) * dinv


def _make_final_body(num_graphs):
    def final_body(p0_ref, p1_ref, y1_ref, dinv_ref, b_ref, bat_ref, out_ref):
        @pl.when(pl.program_id(0) == 0)
        def _():
            out_ref[:] = jnp.full(out_ref.shape, -jnp.inf, jnp.float32)

        pre = (p0_ref[:] + p1_ref[:] + y1_ref[:]) * dinv_ref[:] + b_ref[:]
        h = jnp.where(pre >= 0, pre, 0.01 * pre)
        bat = bat_ref[:]
        for g in range(num_graphs):
            m = jnp.max(jnp.where(bat == g, h, -jnp.inf), axis=0, keepdims=True)
            out_ref[g:g + 1, :] = jnp.maximum(out_ref[g:g + 1, :], m)

    return final_body


def _row_spec(bn, width):
    return pl.BlockSpec((bn, width), lambda i: (i, 0))


def _full_spec(shape):
    return pl.BlockSpec(shape, lambda i: tuple(0 for _ in shape))


def kernel(x, edge_index, batch, W0, b0, W1, b1):
    n, in_ch = x.shape
    hid = W0.shape[1]
    out_ch = W1.shape[1]
    e = edge_index.shape[1]
    num_graphs = 64

    n_pad = ((n + 1 + NS * 16 - 1) // (NS * 16)) * (NS * 16)
    cpt = -(-e // (CH * NC * NS))
    cpt = ((cpt + 7) // 8) * 8
    e_pad = cpt * CH * NC * NS

    pad = jnp.full((e_pad - e,), n, jnp.int32)
    src = jnp.concatenate([edge_index[0], pad]).reshape(e_pad // CH, CH)
    dst = jnp.concatenate([edge_index[1], pad]).reshape(e_pad // CH, CH)

    ones_col = jnp.ones((CH, 1), jnp.float32)
    z1 = jnp.zeros((n_pad, 1), jnp.float32)
    zh = jnp.zeros((n_pad, hid), jnp.float32)
    zo = jnp.zeros((n_pad, out_ch), jnp.float32)

    degp = _make_deg(n_pad, cpt)(dst, ones_col, z1)

    bn = 1000
    grid = (n // bn,)

    y0, dinv = pl.pallas_call(
        _mm0_body,
        grid=grid,
        in_specs=[
            _row_spec(bn, in_ch),
            _full_spec((in_ch, hid)),
            _row_spec(bn, 1),
            _row_spec(bn, 1),
        ],
        out_specs=[_row_spec(bn, hid), _row_spec(bn, 1)],
        out_shape=[
            jax.ShapeDtypeStruct((n, hid), jnp.float32),
            jax.ShapeDtypeStruct((n, 1), jnp.float32),
        ],
    )(x, W0, degp[0, :n], degp[1, :n])

    y0_pad = jnp.pad(y0, ((0, n_pad - n), (0, 0)))
    p = _make_agg(n_pad, cpt, hid)(y0_pad, src, dst, zh)

    y1 = pl.pallas_call(
        _mid_body,
        grid=grid,
        in_specs=[
            _row_spec(bn, hid),
            _row_spec(bn, hid),
            _row_spec(bn, hid),
            _row_spec(bn, 1),
            _full_spec((1, hid)),
            _full_spec((hid, out_ch)),
        ],
        out_specs=_row_spec(bn, out_ch),
        out_shape=jax.ShapeDtypeStruct((n, out_ch), jnp.float32),
    )(p[0, :n], p[1, :n], y0, dinv, b0[None, :], W1)

    y1_pad = jnp.pad(y1, ((0, n_pad - n), (0, 0)))
    p2 = _make_agg(n_pad, cpt, out_ch)(y1_pad, src, dst, zo)

    out = pl.pallas_call(
        _make_final_body(num_graphs),
        grid=grid,
        in_specs=[
            _row_spec(bn, out_ch),
            _row_spec(bn, out_ch),
            _row_spec(bn, out_ch),
            _row_spec(bn, 1),
            _full_spec((1, out_ch)),
            _row_spec(bn, 1),
        ],
        out_specs=_full_spec((num_graphs, out_ch)),
        out_shape=jax.ShapeDtypeStruct((num_graphs, out_ch), jnp.float32),
    )(p2[0, :n], p2[1, :n], y1, dinv, b1[None, :], batch[:, None])

    return out


# speedup vs baseline: 18.3383x; 1.1802x over previous
import functools

import jax
import jax.numpy as jnp
from jax import lax
from jax.experimental import pallas as pl
from jax.experimental.pallas import tpu as pltpu
from jax.experimental.pallas import tpu_sc as plsc

NC = 2
NS = 16
CH = 128


def _make_deg(n_pad, cpt):
    rows_per_tile = n_pad // NS
    mesh = plsc.VectorSubcoreMesh(core_axis_name="c", subcore_axis_name="s")

    @functools.partial(
        pl.kernel,
        out_type=jax.ShapeDtypeStruct((NC, n_pad, 1), jnp.float32),
        mesh=mesh,
        scratch_types=[
            pltpu.VMEM((cpt, CH), jnp.int32),
            pltpu.VMEM((CH, 1), jnp.float32),
            pltpu.VMEM_SHARED((n_pad, 1), jnp.float32),
            pltpu.SemaphoreType.DMA,
        ],
        compiler_params=pltpu.CompilerParams(use_tc_tiling_on_sc=False),
    )
    def deg_kernel(dst_hbm, ones_hbm, zeros_hbm, out_hbm, dst_v, ones_v, acc,
                   ssem):
        c = lax.axis_index("c")
        s = lax.axis_index("s")
        wid = s * NC + c
        r0 = s * rows_per_tile
        pltpu.sync_copy(zeros_hbm.at[pl.ds(r0, rows_per_tile)],
                        acc.at[pl.ds(r0, rows_per_tile)])
        pltpu.sync_copy(dst_hbm.at[pl.ds(wid * cpt, cpt)], dst_v)
        pltpu.sync_copy(ones_hbm, ones_v)
        plsc.subcore_barrier()

        def body(j, carry):
            pltpu.async_copy(ones_v, acc.at[dst_v.at[j]], ssem, add=True)
            return carry

        lax.fori_loop(0, cpt, body, 0)

        def drain(j, carry):
            pltpu.make_async_copy(ones_v, acc.at[dst_v.at[j]], ssem).wait()
            return carry

        lax.fori_loop(0, cpt, drain, 0)
        plsc.subcore_barrier()
        pltpu.sync_copy(acc.at[pl.ds(r0, rows_per_tile)],
                        out_hbm.at[c, pl.ds(r0, rows_per_tile)])

    return deg_kernel


def _make_agg(n_pad, cpt, width):
    rows_per_tile = n_pad // NS
    mesh = plsc.VectorSubcoreMesh(core_axis_name="c", subcore_axis_name="s")

    nbuf = 8
    assert cpt % nbuf == 0
    rounds = cpt // nbuf

    @functools.partial(
        pl.kernel,
        out_type=jax.ShapeDtypeStruct((NC, n_pad, width), jnp.float32),
        mesh=mesh,
        scratch_types=[
            pltpu.VMEM((cpt, CH), jnp.int32),
            pltpu.VMEM((cpt, CH), jnp.int32),
            pltpu.VMEM((nbuf, CH, width), jnp.float32),
            pltpu.VMEM_SHARED((n_pad, width), jnp.float32),
            pltpu.SemaphoreType.DMA((nbuf,)),
            pltpu.SemaphoreType.DMA((nbuf,)),
        ],
        compiler_params=pltpu.CompilerParams(use_tc_tiling_on_sc=False),
    )
    def agg_kernel(table_hbm, src_hbm, dst_hbm, zeros_hbm, out_hbm,
                   src_v, dst_v, rows_v, acc, gsem, ssem):
        c = lax.axis_index("c")
        s = lax.axis_index("s")
        wid = s * NC + c
        r0 = s * rows_per_tile
        pltpu.sync_copy(src_hbm.at[pl.ds(wid * cpt, cpt)], src_v)
        pltpu.sync_copy(dst_hbm.at[pl.ds(wid * cpt, cpt)], dst_v)
        for b in range(nbuf):
            pltpu.async_copy(table_hbm.at[src_v.at[b]], rows_v.at[b],
                             gsem.at[b])
        pltpu.sync_copy(zeros_hbm.at[pl.ds(r0, rows_per_tile)],
                        acc.at[pl.ds(r0, rows_per_tile)])
        plsc.subcore_barrier()

        def body(g, carry):
            base = g * nbuf
            for b in range(nbuf):
                pltpu.make_async_copy(table_hbm.at[src_v.at[base + b]],
                                      rows_v.at[b], gsem.at[b]).wait()
                pltpu.async_copy(rows_v.at[b], acc.at[dst_v.at[base + b]],
                                 ssem.at[b], add=True)
            for b in range(nbuf):
                pltpu.make_async_copy(rows_v.at[b],
                                      acc.at[dst_v.at[base + b]],
                                      ssem.at[b]).wait()
                pltpu.async_copy(table_hbm.at[src_v.at[base + nbuf + b]],
                                 rows_v.at[b], gsem.at[b])
            return carry

        lax.fori_loop(0, rounds - 1, body, 0)

        base = (rounds - 1) * nbuf
        for b in range(nbuf):
            pltpu.make_async_copy(table_hbm.at[src_v.at[base + b]],
                                  rows_v.at[b], gsem.at[b]).wait()
            pltpu.async_copy(rows_v.at[b], acc.at[dst_v.at[base + b]],
                             ssem.at[b], add=True)
        for b in range(nbuf):
            pltpu.make_async_copy(rows_v.at[b], acc.at[dst_v.at[base + b]],
                                  ssem.at[b]).wait()
        plsc.subcore_barrier()
        pltpu.sync_copy(acc.at[pl.ds(r0, rows_per_tile)],
                        out_hbm.at[c, pl.ds(r0, rows_per_tile)])

    return agg_kernel


def _mm0_body(x_ref, w_ref, p0_ref, p1_ref, y_ref, dinv_ref):
    deg = p0_ref[:] + p1_ref[:] + 1.0
    dinv = lax.rsqrt(deg)
    y = jnp.dot(x_ref[:], w_ref[:], preferred_element_type=jnp.float32)
    y_ref[:] = y * dinv
    dinv_ref[:] = dinv


def _mid_body(p0_ref, p1_ref, y0_ref, dinv_ref, b_ref, w_ref, out_ref):
    dinv = dinv_ref[:]
    pre = (p0_ref[:] + p1_ref[:] + y0_ref[:]) * dinv + b_ref[:]
    h = jnp.where(pre >= 0, pre, 0.01 * pre)
    out_ref[:] = jnp.dot(h, w_ref[:], preferred_element_type=jnp.float32) * dinv


def _make_final_body(num_graphs):
    def final_body(p0_ref, p1_ref, y1_ref, dinv_ref, b_ref, bat_ref, out_ref):
        @pl.when(pl.program_id(0) == 0)
        def _():
            out_ref[:] = jnp.full(out_ref.shape, -jnp.inf, jnp.float32)

        pre = (p0_ref[:] + p1_ref[:] + y1_ref[:]) * dinv_ref[:] + b_ref[:]
        h = jnp.where(pre >= 0, pre, 0.01 * pre)
        bat = bat_ref[:]
        for g in range(num_graphs):
            m = jnp.max(jnp.where(bat == g, h, -jnp.inf), axis=0, keepdims=True)
            out_ref[g:g + 1, :] = jnp.maximum(out_ref[g:g + 1, :], m)

    return final_body


def _row_spec(bn, width):
    return pl.BlockSpec((bn, width), lambda i: (i, 0))


def _full_spec(shape):
    return pl.BlockSpec(shape, lambda i: tuple(0 for _ in shape))


def kernel(x, edge_index, batch, W0, b0, W1, b1):
    n, in_ch = x.shape
    hid = W0.shape[1]
    out_ch = W1.shape[1]
    e = edge_index.shape[1]
    num_graphs = 64

    n_pad = ((n + 1 + NS * 16 - 1) // (NS * 16)) * (NS * 16)
    cpt = -(-e // (CH * NC * NS))
    cpt = ((cpt + 7) // 8) * 8
    e_pad = cpt * CH * NC * NS

    pad = jnp.full((e_pad - e,), n, jnp.int32)
    src = jnp.concatenate([edge_index[0], pad]).reshape(e_pad // CH, CH)
    dst = jnp.concatenate([edge_index[1], pad]).reshape(e_pad // CH, CH)

    ones_col = jnp.ones((CH, 1), jnp.float32)
    z1 = jnp.zeros((n_pad, 1), jnp.float32)
    zh = jnp.zeros((n_pad, hid), jnp.float32)
    zo = jnp.zeros((n_pad, out_ch), jnp.float32)

    degp = _make_deg(n_pad, cpt)(dst, ones_col, z1)

    bn = 1000
    grid = (n // bn,)

    y0, dinv = pl.pallas_call(
        _mm0_body,
        grid=grid,
        in_specs=[
            _row_spec(bn, in_ch),
            _full_spec((in_ch, hid)),
            _row_spec(bn, 1),
            _row_spec(bn, 1),
        ],
        out_specs=[_row_spec(bn, hid), _row_spec(bn, 1)],
        out_shape=[
            jax.ShapeDtypeStruct((n, hid), jnp.float32),
            jax.ShapeDtypeStruct((n, 1), jnp.float32),
        ],
    )(x, W0, degp[0, :n], degp[1, :n])

    y0_pad = jnp.pad(y0, ((0, n_pad - n), (0, 0)))
    p = _make_agg(n_pad, cpt, hid)(y0_pad, src, dst, zh)

    y1 = pl.pallas_call(
        _mid_body,
        grid=grid,
        in_specs=[
            _row_spec(bn, hid),
            _row_spec(bn, hid),
            _row_spec(bn, hid),
            _row_spec(bn, 1),
            _full_spec((1, hid)),
            _full_spec((hid, out_ch)),
        ],
        out_specs=_row_spec(bn, out_ch),
        out_shape=jax.ShapeDtypeStruct((n, out_ch), jnp.float32),
    )(p[0, :n], p[1, :n], y0, dinv, b0[None, :], W1)

    y1_pad = jnp.pad(y1, ((0, n_pad - n), (0, 0)))
    p2 = _make_agg(n_pad, cpt, out_ch)(y1_pad, src, dst, zo)

    out = pl.pallas_call(
        _make_final_body(num_graphs),
        grid=grid,
        in_specs=[
            _row_spec(bn, out_ch),
            _row_spec(bn, out_ch),
            _row_spec(bn, out_ch),
            _row_spec(bn, 1),
            _full_spec((1, out_ch)),
            _row_spec(bn, 1),
        ],
        out_specs=_full_spec((num_graphs, out_ch)),
        out_shape=jax.ShapeDtypeStruct((num_graphs, out_ch), jnp.float32),
    )(p2[0, :n], p2[1, :n], y1, dinv, b1[None, :], batch[:, None])

    return out


# speedup vs baseline: 18.8509x; 1.0280x over previous
import functools

import jax
import jax.numpy as jnp
from jax import lax
from jax.experimental import pallas as pl
from jax.experimental.pallas import tpu as pltpu
from jax.experimental.pallas import tpu_sc as plsc

NC = 2
NS = 16
CH = 128


def _make_deg(n_pad, cpt0, cpt1):
    rows_per_tile = n_pad // NS
    mesh = plsc.VectorSubcoreMesh(core_axis_name="c", subcore_axis_name="s")

    @functools.partial(
        pl.kernel,
        out_type=jax.ShapeDtypeStruct((NC, n_pad, 16), jnp.float32),
        mesh=mesh,
        scratch_types=[
            pltpu.VMEM((cpt0, CH), jnp.int32),
            pltpu.VMEM((CH, 16), jnp.float32),
            pltpu.VMEM_SHARED((n_pad, 16), jnp.float32),
        ],
        compiler_params=pltpu.CompilerParams(use_tc_tiling_on_sc=False),
    )
    def deg_kernel(dst_hbm, ones_hbm, zeros_hbm, out_hbm, dst_v, ones_v, acc):
        c = lax.axis_index("c")
        s = lax.axis_index("s")
        r0 = s * rows_per_tile
        cpt_c = jnp.where(c == 0, cpt0, cpt1)
        start = jnp.where(c == 0, s * cpt0, NS * cpt0 + s * cpt1)
        pltpu.sync_copy(zeros_hbm.at[pl.ds(r0, rows_per_tile)],
                        acc.at[pl.ds(r0, rows_per_tile)])

        @pl.when(c == 0)
        def _():
            pltpu.sync_copy(dst_hbm.at[pl.ds(start, cpt0)], dst_v)

        @pl.when(c != 0)
        def _():
            pltpu.sync_copy(dst_hbm.at[pl.ds(start, cpt1)],
                            dst_v.at[pl.ds(0, cpt1)])

        pltpu.sync_copy(ones_hbm, ones_v)
        plsc.subcore_barrier()

        def body(j, carry):
            pltpu.sync_copy(ones_v, acc.at[dst_v.at[j]], add=True)
            return carry

        lax.fori_loop(0, cpt_c, body, 0)
        plsc.subcore_barrier()
        pltpu.sync_copy(acc.at[pl.ds(r0, rows_per_tile)],
                        out_hbm.at[c, pl.ds(r0, rows_per_tile)])

    return deg_kernel


def _make_agg(n_pad, width, ch, cpt0, cpt1):
    rows_per_tile = n_pad // NS
    mesh = plsc.VectorSubcoreMesh(core_axis_name="c", subcore_axis_name="s")

    nbuf = 8
    assert cpt0 % nbuf == 0 and cpt1 % nbuf == 0

    @functools.partial(
        pl.kernel,
        out_type=jax.ShapeDtypeStruct((NC, n_pad, width), jnp.float32),
        mesh=mesh,
        scratch_types=[
            pltpu.VMEM((cpt0, ch), jnp.int32),
            pltpu.VMEM((cpt0, ch), jnp.int32),
            pltpu.VMEM((nbuf, ch, width), jnp.float32),
            pltpu.VMEM_SHARED((n_pad, width), jnp.float32),
            pltpu.SemaphoreType.DMA((nbuf,)),
            pltpu.SemaphoreType.DMA((nbuf,)),
        ],
        compiler_params=pltpu.CompilerParams(use_tc_tiling_on_sc=False),
    )
    def agg_kernel(table_hbm, src_hbm, dst_hbm, zeros_hbm, out_hbm,
                   src_v, dst_v, rows_v, acc, gsem, ssem):
        c = lax.axis_index("c")
        s = lax.axis_index("s")
        r0 = s * rows_per_tile
        rounds = jnp.where(c == 0, cpt0 // nbuf, cpt1 // nbuf)
        start = jnp.where(c == 0, s * cpt0, NS * cpt0 + s * cpt1)

        @pl.when(c == 0)
        def _():
            pltpu.sync_copy(src_hbm.at[pl.ds(start, cpt0)], src_v)
            pltpu.sync_copy(dst_hbm.at[pl.ds(start, cpt0)], dst_v)

        @pl.when(c != 0)
        def _():
            pltpu.sync_copy(src_hbm.at[pl.ds(start, cpt1)],
                            src_v.at[pl.ds(0, cpt1)])
            pltpu.sync_copy(dst_hbm.at[pl.ds(start, cpt1)],
                            dst_v.at[pl.ds(0, cpt1)])

        for b in range(nbuf):
            pltpu.async_copy(table_hbm.at[src_v.at[b]], rows_v.at[b],
                             gsem.at[b])
        pltpu.sync_copy(zeros_hbm.at[pl.ds(r0, rows_per_tile)],
                        acc.at[pl.ds(r0, rows_per_tile)])
        plsc.subcore_barrier()

        def body(g, carry):
            base = g * nbuf
            for b in range(nbuf):
                pltpu.make_async_copy(table_hbm.at[src_v.at[base + b]],
                                      rows_v.at[b], gsem.at[b]).wait()
                pltpu.async_copy(rows_v.at[b], acc.at[dst_v.at[base + b]],
                                 ssem.at[b], add=True)
            for b in range(nbuf):
                pltpu.make_async_copy(rows_v.at[b],
                                      acc.at[dst_v.at[base + b]],
                                      ssem.at[b]).wait()
                pltpu.async_copy(table_hbm.at[src_v.at[base + nbuf + b]],
                                 rows_v.at[b], gsem.at[b])
            return carry

        lax.fori_loop(0, rounds - 1, body, 0)

        base = (rounds - 1) * nbuf
        for b in range(nbuf):
            pltpu.make_async_copy(table_hbm.at[src_v.at[base + b]],
                                  rows_v.at[b], gsem.at[b]).wait()
            pltpu.async_copy(rows_v.at[b], acc.at[dst_v.at[base + b]],
                             ssem.at[b], add=True)
        for b in range(nbuf):
            pltpu.make_async_copy(rows_v.at[b], acc.at[dst_v.at[base + b]],
                                  ssem.at[b]).wait()
        plsc.subcore_barrier()
        pltpu.sync_copy(acc.at[pl.ds(r0, rows_per_tile)],
                        out_hbm.at[c, pl.ds(r0, rows_per_tile)])

    return agg_kernel


def _mm0_body(x_ref, w_ref, p0_ref, p1_ref, y_ref, dinv_ref):
    deg = p0_ref[:] + p1_ref[:] + 1.0
    dinv = lax.rsqrt(deg)
    y = jnp.dot(x_ref[:], w_ref[:], preferred_element_type=jnp.float32)
    y_ref[:] = y * dinv
    dinv_ref[:] = dinv


def _mid_body(p0_ref, p1_ref, y0_ref, dinv_ref, b_ref, w_ref, out_ref):
    dinv = dinv_ref[:]
    pre = (p0_ref[:] + p1_ref[:] + y0_ref[:]) * dinv + b_ref[:]
    h = jnp.where(pre >= 0, pre, 0.01 * pre)
    out_ref[:] = jnp.dot(h, w_ref[:], preferred_element_type=jnp.float32) * dinv


def _make_final_body(num_graphs):
    def final_body(p0_ref, p1_ref, y1_ref, dinv_ref, b_ref, bat_ref, out_ref):
        @pl.when(pl.program_id(0) == 0)
        def _():
            out_ref[:] = jnp.full(out_ref.shape, -jnp.inf, jnp.float32)

        pre = (p0_ref[:] + p1_ref[:] + y1_ref[:]) * dinv_ref[:] + b_ref[:]
        h = jnp.where(pre >= 0, pre, 0.01 * pre)
        bat = bat_ref[:]
        for g in range(num_graphs):
            m = jnp.max(jnp.where(bat == g, h, -jnp.inf), axis=0, keepdims=True)
            out_ref[g:g + 1, :] = jnp.maximum(out_ref[g:g + 1, :], m)

    return final_body


def _row_spec(bn, width):
    return pl.BlockSpec((bn, width), lambda i: (i, 0))


def _full_spec(shape):
    return pl.BlockSpec(shape, lambda i: tuple(0 for _ in shape))


def kernel(x, edge_index, batch, W0, b0, W1, b1):
    n, in_ch = x.shape
    hid = W0.shape[1]
    out_ch = W1.shape[1]
    e = edge_index.shape[1]
    num_graphs = 64

    n_pad = ((n + 1 + NS * 16 - 1) // (NS * 16)) * (NS * 16)
    e_pad = ((e + NS * 8 * 128 - 1) // (NS * 8 * 128)) * (NS * 8 * 128)

    def split(tot):
        c0 = min(int(round(tot * 0.8 / 8)) * 8, tot - 8)
        return c0, tot - c0

    cpt0_128, cpt1_128 = split(e_pad // (128 * NS))
    cpt0_64, cpt1_64 = split(e_pad // (64 * NS))

    pad = jnp.full((e_pad - e,), n, jnp.int32)
    src_flat = jnp.concatenate([edge_index[0], pad])
    dst_flat = jnp.concatenate([edge_index[1], pad])
    src64 = src_flat.reshape(e_pad // 64, 64)
    dst64 = dst_flat.reshape(e_pad // 64, 64)
    src128 = src_flat.reshape(e_pad // 128, 128)
    dst128 = dst_flat.reshape(e_pad // 128, 128)

    ones_col = jnp.ones((CH, 16), jnp.float32)
    z1 = jnp.zeros((n_pad, 16), jnp.float32)
    zh = jnp.zeros((n_pad, hid), jnp.float32)
    zo = jnp.zeros((n_pad, out_ch), jnp.float32)

    degp = _make_deg(n_pad, cpt0_128, cpt1_128)(dst128, ones_col, z1)

    bn = 1000
    grid = (n // bn,)

    y0, dinv = pl.pallas_call(
        _mm0_body,
        grid=grid,
        in_specs=[
            _row_spec(bn, in_ch),
            _full_spec((in_ch, hid)),
            _row_spec(bn, 1),
            _row_spec(bn, 1),
        ],
        out_specs=[_row_spec(bn, hid), _row_spec(bn, 1)],
        out_shape=[
            jax.ShapeDtypeStruct((n, hid), jnp.float32),
            jax.ShapeDtypeStruct((n, 1), jnp.float32),
        ],
    )(x, W0, degp[0, :n, 0:1], degp[1, :n, 0:1])

    y0_pad = jnp.pad(y0, ((0, n_pad - n), (0, 0)))
    p = _make_agg(n_pad, hid, 64, cpt0_64, cpt1_64)(y0_pad, src64, dst64, zh)

    y1 = pl.pallas_call(
        _mid_body,
        grid=grid,
        in_specs=[
            _row_spec(bn, hid),
            _row_spec(bn, hid),
            _row_spec(bn, hid),
            _row_spec(bn, 1),
            _full_spec((1, hid)),
            _full_spec((hid, out_ch)),
        ],
        out_specs=_row_spec(bn, out_ch),
        out_shape=jax.ShapeDtypeStruct((n, out_ch), jnp.float32),
    )(p[0, :n], p[1, :n], y0, dinv, b0[None, :], W1)

    y1_pad = jnp.pad(y1, ((0, n_pad - n), (0, 0)))
    p2 = _make_agg(n_pad, out_ch, 128, cpt0_128, cpt1_128)(y1_pad, src128, dst128, zo)

    out = pl.pallas_call(
        _make_final_body(num_graphs),
        grid=grid,
        in_specs=[
            _row_spec(bn, out_ch),
            _row_spec(bn, out_ch),
            _row_spec(bn, out_ch),
            _row_spec(bn, 1),
            _full_spec((1, out_ch)),
            _row_spec(bn, 1),
        ],
        out_specs=_full_spec((num_graphs, out_ch)),
        out_shape=jax.ShapeDtypeStruct((num_graphs, out_ch), jnp.float32),
    )(p2[0, :n], p2[1, :n], y1, dinv, b1[None, :], batch[:, None])

    return out
